# baseline (speedup 1.0000x reference)
import jax
import jax.numpy as jnp
from jax.experimental import pallas as pl

_BM = 400


def _mm_block(adj_ref, emb_ref, out_ref):
    out_ref[...] = jnp.dot(adj_ref[...], emb_ref[...],
                           preferred_element_type=jnp.float32)


def kernel(adj, embeds):
    m, k = adj.shape
    n = embeds.shape[1]
    return pl.pallas_call(
        _mm_block,
        grid=(m // _BM,),
        in_specs=[
            pl.BlockSpec((_BM, k), lambda i: (i, 0)),
            pl.BlockSpec((k, n), lambda i: (0, 0)),
        ],
        out_specs=pl.BlockSpec((_BM, n), lambda i: (i, 0)),
        out_shape=jax.ShapeDtypeStruct((m, n), jnp.float32),
    )(adj, embeds)


# speedup vs baseline: 1.0003x; 1.0003x over previous
import jax
import jax.numpy as jnp
from jax.experimental import pallas as pl

_BM = 400


def _mm_block(adj_ref, emb_ref, out_ref):
    out_ref[...] = jax.lax.dot_general(
        adj_ref[...], emb_ref[...],
        dimension_numbers=(((1,), (0,)), ((), ())),
        precision=jax.lax.Precision.DEFAULT,
        preferred_element_type=jnp.float32)


def kernel(adj, embeds):
    m, k = adj.shape
    n = embeds.shape[1]
    return pl.pallas_call(
        _mm_block,
        grid=(m // _BM,),
        in_specs=[
            pl.BlockSpec((_BM, k), lambda i: (i, 0)),
            pl.BlockSpec((k, n), lambda i: (0, 0)),
        ],
        out_specs=pl.BlockSpec((_BM, n), lambda i: (i, 0)),
        out_shape=jax.ShapeDtypeStruct((m, n), jnp.float32),
    )(adj, embeds)


# speedup vs baseline: 1.0128x; 1.0125x over previous
import jax
import jax.numpy as jnp
from jax.experimental import pallas as pl

_BM = 200


def _mm_block(adj_ref, emb_ref, out_ref):
    out_ref[...] = jax.lax.dot_general(
        adj_ref[...], emb_ref[...],
        dimension_numbers=(((1,), (0,)), ((), ())),
        precision=jax.lax.Precision.DEFAULT,
        preferred_element_type=jnp.float32)


def kernel(adj, embeds):
    m, k = adj.shape
    n = embeds.shape[1]
    return pl.pallas_call(
        _mm_block,
        grid=(m // _BM,),
        in_specs=[
            pl.BlockSpec((_BM, k), lambda i: (i, 0)),
            pl.BlockSpec((k, n), lambda i: (0, 0)),
        ],
        out_specs=pl.BlockSpec((_BM, n), lambda i: (i, 0)),
        out_shape=jax.ShapeDtypeStruct((m, n), jnp.float32),
    )(adj, embeds)
